# Initial kernel scaffold; baseline (speedup 1.0000x reference)
import jax
import jax.numpy as jnp
from jax.experimental import pallas as pl


def kernel(edge_index, edge_vals, u_f, v_f, W):
    raise NotImplementedError("write your pallas kernel here")


# speedup vs baseline: 3.5600x; 3.5600x over previous
import functools

import jax
import jax.numpy as jnp
from jax import lax
from jax.experimental import pallas as pl
from jax.experimental.pallas import tpu as pltpu
from jax.experimental.pallas import tpu_sc as plsc

N_NODES = 10000
D_IN = 128
D_OUT = 128

_NC = 2
_NS = 16
_NW = _NC * _NS
_CPW = D_OUT // _NW
_LANES = 16
_CHUNK = 2000


def _mm_body(nf_ref, w_ref, out_ref):
    out_ref[...] = lax.dot_general(
        w_ref[...], nf_ref[...],
        dimension_numbers=(((0,), (1,)), ((), ())),
        preferred_element_type=jnp.float32,
    )


def _matmul_t(node_f, W):
    n = node_f.shape[0]
    blk = 2048
    grid = pl.cdiv(n, blk)
    return pl.pallas_call(
        _mm_body,
        grid=(grid,),
        in_specs=[
            pl.BlockSpec((blk, D_IN), lambda i: (i, 0)),
            pl.BlockSpec((D_IN, D_OUT), lambda i: (0, 0)),
        ],
        out_specs=pl.BlockSpec((D_OUT, blk), lambda i: (0, i)),
        out_shape=jax.ShapeDtypeStruct((D_OUT, n), jnp.float32),
    )(node_f, W)


def _spmm_body(hT_hbm, row_hbm, col_hbm, val_hbm, out_hbm,
               h0, h1, h2, h3, o0, o1, o2, o3, rb, cb, vb):
    h_bufs = (h0, h1, h2, h3)
    o_bufs = (o0, o1, o2, o3)
    n_edges = row_hbm.shape[0]
    n_chunks = n_edges // _CHUNK
    groups_per_chunk = _CHUNK // _LANES

    sid = lax.axis_index("s")
    cid = lax.axis_index("c")
    wid = sid * _NC + cid
    col_base = wid * _CPW

    for j in range(_CPW):
        pltpu.sync_copy(hT_hbm.at[col_base + j], h_bufs[j])

    zeros16 = jnp.zeros((_LANES,), jnp.float32)

    @plsc.parallel_loop(0, N_NODES // _LANES, unroll=8)
    def _zero(i):
        off = pl.multiple_of(i * _LANES, _LANES)
        for j in range(_CPW):
            o_bufs[j][pl.ds(off, _LANES)] = zeros16

    def chunk_body(g, _):
        base = g * _CHUNK
        pltpu.sync_copy(row_hbm.at[pl.ds(base, _CHUNK)], rb)
        pltpu.sync_copy(col_hbm.at[pl.ds(base, _CHUNK)], cb)
        pltpu.sync_copy(val_hbm.at[pl.ds(base, _CHUNK)], vb)

        @plsc.parallel_loop(0, groups_per_chunk, unroll=4)
        def _groups(i):
            off = pl.multiple_of(i * _LANES, _LANES)
            r16 = rb[pl.ds(off, _LANES)]
            c16 = cb[pl.ds(off, _LANES)]
            v16 = vb[pl.ds(off, _LANES)]
            for j in range(_CPW):
                x = plsc.load_gather(h_bufs[j], [c16])
                plsc.addupdate_scatter(o_bufs[j], [r16], x * v16)

        return _

    lax.fori_loop(0, n_chunks, chunk_body, 0)

    for j in range(_CPW):
        pltpu.sync_copy(o_bufs[j], out_hbm.at[col_base + j])


def _spmm(hT, row, col, val):
    mesh = plsc.VectorSubcoreMesh(core_axis_name="c", subcore_axis_name="s")
    f = pl.kernel(
        _spmm_body,
        out_type=jax.ShapeDtypeStruct((D_OUT, N_NODES), jnp.float32),
        mesh=mesh,
        compiler_params=pltpu.CompilerParams(needs_layout_passes=False),
        scratch_types=(
            [pltpu.VMEM((N_NODES,), jnp.float32) for _ in range(2 * _CPW)]
            + [
                pltpu.VMEM((_CHUNK,), jnp.int32),
                pltpu.VMEM((_CHUNK,), jnp.int32),
                pltpu.VMEM((_CHUNK,), jnp.float32),
            ]
        ),
    )
    return f(hT, row, col, val)


def kernel(edge_index, edge_vals, u_f, v_f, W):
    node_f = jnp.concatenate([u_f, v_f], axis=0)
    hT = _matmul_t(node_f, W.astype(jnp.float32))
    row = edge_index[0].astype(jnp.int32)
    col = edge_index[1].astype(jnp.int32)
    outT = _spmm(hT, row, col, edge_vals.astype(jnp.float32))
    return outT.T


# speedup vs baseline: 7.6268x; 2.1424x over previous
import functools

import jax
import jax.numpy as jnp
from jax import lax
from jax.experimental import pallas as pl
from jax.experimental.pallas import tpu as pltpu
from jax.experimental.pallas import tpu_sc as plsc

N_NODES = 10000
D_IN = 128
D_OUT = 128

_NC = 2
_NS = 16
_NW = _NC * _NS
_CPW = D_OUT // _NW
_LANES = 16
_CHUNK = 4000


def _mm_body(nf_ref, w_ref, out_ref):
    out_ref[...] = lax.dot_general(
        w_ref[...], nf_ref[...],
        dimension_numbers=(((0,), (1,)), ((), ())),
        preferred_element_type=jnp.float32,
    )


def _matmul_t(node_f, W):
    n = node_f.shape[0]
    blk = 2048
    grid = pl.cdiv(n, blk)
    return pl.pallas_call(
        _mm_body,
        grid=(grid,),
        in_specs=[
            pl.BlockSpec((blk, D_IN), lambda i: (i, 0)),
            pl.BlockSpec((D_IN, D_OUT), lambda i: (0, 0)),
        ],
        out_specs=pl.BlockSpec((D_OUT, blk), lambda i: (0, i)),
        out_shape=jax.ShapeDtypeStruct((D_OUT, n), jnp.float32),
    )(node_f, W)


def _spmm_body(hT_hbm, row_hbm, col_hbm, val_hbm, out_hbm,
               h0, h1, h2, h3, o0, o1, o2, o3,
               rb0, cb0, vb0, rb1, cb1, vb1, sem0, sem1):
    h_bufs = (h0, h1, h2, h3)
    o_bufs = (o0, o1, o2, o3)
    ebufs = ((rb0, cb0, vb0), (rb1, cb1, vb1))
    sems = (sem0, sem1)
    n_edges = row_hbm.shape[0]
    n_chunks = n_edges // _CHUNK
    groups_per_chunk = _CHUNK // _LANES

    sid = lax.axis_index("s")
    cid = lax.axis_index("c")
    wid = sid * _NC + cid
    col_base = wid * _CPW

    for j in range(_CPW):
        pltpu.sync_copy(hT_hbm.at[col_base + j], h_bufs[j])

    zeros16 = jnp.zeros((_LANES,), jnp.float32)

    @plsc.parallel_loop(0, N_NODES // _LANES, unroll=8)
    def _zero(i):
        off = pl.multiple_of(i * _LANES, _LANES)
        for j in range(_CPW):
            o_bufs[j][pl.ds(off, _LANES)] = zeros16

    def _issue(g, slot):
        base = pl.multiple_of(g * _CHUNK, 8)
        rb, cb, vb = ebufs[slot]
        sem = sems[slot]
        pltpu.async_copy(row_hbm.at[pl.ds(base, _CHUNK)], rb, sem)
        pltpu.async_copy(col_hbm.at[pl.ds(base, _CHUNK)], cb, sem)
        pltpu.async_copy(val_hbm.at[pl.ds(base, _CHUNK)], vb, sem)

    def _drain(g, slot):
        base = pl.multiple_of(g * _CHUNK, 8)
        rb, cb, vb = ebufs[slot]
        sem = sems[slot]
        pltpu.make_async_copy(row_hbm.at[pl.ds(base, _CHUNK)], rb, sem).wait()
        pltpu.make_async_copy(col_hbm.at[pl.ds(base, _CHUNK)], cb, sem).wait()
        pltpu.make_async_copy(val_hbm.at[pl.ds(base, _CHUNK)], vb, sem).wait()

    def _process(slot):
        rb, cb, vb = ebufs[slot]

        @plsc.parallel_loop(0, groups_per_chunk, unroll=8)
        def _groups(i):
            off = pl.multiple_of(i * _LANES, _LANES)
            r16 = rb[pl.ds(off, _LANES)]
            c16 = cb[pl.ds(off, _LANES)]
            v16 = vb[pl.ds(off, _LANES)]
            for j in range(_CPW):
                x = plsc.load_gather(h_bufs[j], [c16])
                plsc.addupdate_scatter(o_bufs[j], [r16], x * v16)

    n_pairs = n_chunks // 2
    _issue(0, 0)
    _issue(1, 1)

    def pair_body(p, carry):
        g0 = p * 2
        _drain(g0, 0)
        _process(0)

        @pl.when(g0 + 2 < n_chunks)
        def _issue0():
            _issue(g0 + 2, 0)

        _drain(g0 + 1, 1)
        _process(1)

        @pl.when(g0 + 3 < n_chunks)
        def _issue1():
            _issue(g0 + 3, 1)

        return carry

    lax.fori_loop(0, n_pairs, pair_body, 0)

    for j in range(_CPW):
        pltpu.sync_copy(o_bufs[j], out_hbm.at[col_base + j])


def _spmm(hT, row, col, val):
    mesh = plsc.VectorSubcoreMesh(core_axis_name="c", subcore_axis_name="s")
    f = pl.kernel(
        _spmm_body,
        out_type=jax.ShapeDtypeStruct((D_OUT, N_NODES), jnp.float32),
        mesh=mesh,
        compiler_params=pltpu.CompilerParams(needs_layout_passes=False),
        scratch_types=(
            [pltpu.VMEM((N_NODES,), jnp.float32) for _ in range(2 * _CPW)]
            + [
                pltpu.VMEM((_CHUNK,), jnp.int32),
                pltpu.VMEM((_CHUNK,), jnp.int32),
                pltpu.VMEM((_CHUNK,), jnp.float32),
                pltpu.VMEM((_CHUNK,), jnp.int32),
                pltpu.VMEM((_CHUNK,), jnp.int32),
                pltpu.VMEM((_CHUNK,), jnp.float32),
                pltpu.SemaphoreType.DMA,
                pltpu.SemaphoreType.DMA,
            ]
        ),
    )
    return f(hT, row, col, val)


def kernel(edge_index, edge_vals, u_f, v_f, W):
    node_f = jnp.concatenate([u_f, v_f], axis=0)
    hT = _matmul_t(node_f, W.astype(jnp.float32))
    row = edge_index[0].astype(jnp.int32)
    col = edge_index[1].astype(jnp.int32)
    outT = _spmm(hT, row, col, edge_vals.astype(jnp.float32))
    return outT.T


# speedup vs baseline: 9.6560x; 1.2661x over previous
import functools

import jax
import jax.numpy as jnp
from jax import lax
from jax.experimental import pallas as pl
from jax.experimental.pallas import tpu as pltpu
from jax.experimental.pallas import tpu_sc as plsc

N_NODES = 10000
D_IN = 128
D_OUT = 128

_NC = 2
_NS = 16
_NW = _NC * _NS
_PPW = 2
_LANES = 16
_CHUNK = 4000


def _mm_pack_body(nf_ref, w_ref, out_ref):
    hT = lax.dot_general(
        w_ref[...], nf_ref[...],
        dimension_numbers=(((0,), (1,)), ((), ())),
        preferred_element_type=jnp.float32,
    )
    lo = lax.bitcast_convert_type(hT[: D_OUT // 2, :], jnp.uint32)
    hi = lax.bitcast_convert_type(hT[D_OUT // 2 :, :], jnp.uint32)
    half = jnp.uint32(0x8000)
    lo16 = (lo + half) >> 16
    hi16 = ((hi + half) >> 16) << 16
    out_ref[...] = lax.bitcast_convert_type(lo16 | hi16, jnp.int32)


def _matmul_pack(node_f, W):
    n = node_f.shape[0]
    blk = 2048
    grid = pl.cdiv(n, blk)
    return pl.pallas_call(
        _mm_pack_body,
        grid=(grid,),
        in_specs=[
            pl.BlockSpec((blk, D_IN), lambda i: (i, 0)),
            pl.BlockSpec((D_IN, D_OUT), lambda i: (0, 0)),
        ],
        out_specs=pl.BlockSpec((D_OUT // 2, blk), lambda i: (0, i)),
        out_shape=jax.ShapeDtypeStruct((D_OUT // 2, n), jnp.int32),
    )(node_f, W)


def _rc_pack_body(ei_ref, out_ref):
    out_ref[...] = ei_ref[0, :] | (ei_ref[1, :] << 16)


def _rc_pack(edge_index):
    e = edge_index.shape[1]
    return pl.pallas_call(
        _rc_pack_body,
        in_specs=[pl.BlockSpec((2, e), lambda: (0, 0))],
        out_specs=pl.BlockSpec((e,), lambda: (0,)),
        out_shape=jax.ShapeDtypeStruct((e,), jnp.int32),
    )(edge_index)


def _spmm_body(hP_hbm, rc_hbm, val_hbm, out_hbm,
               hp0, hp1, o0, o1, o2, o3,
               rb0, vb0, rb1, vb1, sem0, sem1):
    hp_bufs = (hp0, hp1)
    o_bufs = ((o0, o1), (o2, o3))
    ebufs = ((rb0, vb0), (rb1, vb1))
    sems = (sem0, sem1)

    n_edges = rc_hbm.shape[0]
    n_chunks = n_edges // _CHUNK
    groups_per_chunk = _CHUNK // _LANES

    sid = lax.axis_index("s")
    cid = lax.axis_index("c")
    wid = sid * _NC + cid
    row_base = wid * _PPW

    for j in range(_PPW):
        pltpu.sync_copy(hP_hbm.at[row_base + j], hp_bufs[j])

    zeros16 = jnp.zeros((_LANES,), jnp.float32)

    @plsc.parallel_loop(0, N_NODES // _LANES, unroll=8)
    def _zero(i):
        off = pl.multiple_of(i * _LANES, _LANES)
        for j in range(_PPW):
            for k in range(2):
                o_bufs[j][k][pl.ds(off, _LANES)] = zeros16

    def _issue(g, slot):
        base = pl.multiple_of(g * _CHUNK, 8)
        rb, vb = ebufs[slot]
        sem = sems[slot]
        pltpu.async_copy(rc_hbm.at[pl.ds(base, _CHUNK)], rb, sem)
        pltpu.async_copy(val_hbm.at[pl.ds(base, _CHUNK)], vb, sem)

    def _drain(g, slot):
        base = pl.multiple_of(g * _CHUNK, 8)
        rb, vb = ebufs[slot]
        sem = sems[slot]
        pltpu.make_async_copy(rc_hbm.at[pl.ds(base, _CHUNK)], rb, sem).wait()
        pltpu.make_async_copy(val_hbm.at[pl.ds(base, _CHUNK)], vb, sem).wait()

    mask16 = jnp.full((_LANES,), 0xFFFF, jnp.int32)
    maskhi = jnp.full((_LANES,), -65536, jnp.int32)

    def _process(slot):
        rb, vb = ebufs[slot]

        @plsc.parallel_loop(0, groups_per_chunk, unroll=8)
        def _groups(i):
            off = pl.multiple_of(i * _LANES, _LANES)
            rc16 = rb[pl.ds(off, _LANES)]
            v16 = vb[pl.ds(off, _LANES)]
            r16 = rc16 & mask16
            c16 = lax.shift_right_logical(rc16, 16)
            for j in range(_PPW):
                g = plsc.load_gather(hp_bufs[j], [c16])
                x_lo = plsc.bitcast(lax.shift_left(g, 16), jnp.float32)
                x_hi = plsc.bitcast(g & maskhi, jnp.float32)
                plsc.addupdate_scatter(o_bufs[j][0], [r16], x_lo * v16)
                plsc.addupdate_scatter(o_bufs[j][1], [r16], x_hi * v16)

    n_pairs = n_chunks // 2
    _issue(0, 0)
    _issue(1, 1)

    def pair_body(p, carry):
        g0 = p * 2
        _drain(g0, 0)
        _process(0)

        @pl.when(g0 + 2 < n_chunks)
        def _issue0():
            _issue(g0 + 2, 0)

        _drain(g0 + 1, 1)
        _process(1)

        @pl.when(g0 + 3 < n_chunks)
        def _issue1():
            _issue(g0 + 3, 1)

        return carry

    lax.fori_loop(0, n_pairs, pair_body, 0)

    for j in range(_PPW):
        pltpu.sync_copy(o_bufs[j][0], out_hbm.at[row_base + j])
        pltpu.sync_copy(o_bufs[j][1], out_hbm.at[D_OUT // 2 + row_base + j])


def _spmm(hP, rc, val):
    mesh = plsc.VectorSubcoreMesh(core_axis_name="c", subcore_axis_name="s")
    f = pl.kernel(
        _spmm_body,
        out_type=jax.ShapeDtypeStruct((D_OUT, N_NODES), jnp.float32),
        mesh=mesh,
        compiler_params=pltpu.CompilerParams(needs_layout_passes=False),
        scratch_types=(
            [pltpu.VMEM((N_NODES,), jnp.int32) for _ in range(_PPW)]
            + [pltpu.VMEM((N_NODES,), jnp.float32) for _ in range(2 * _PPW)]
            + [
                pltpu.VMEM((_CHUNK,), jnp.int32),
                pltpu.VMEM((_CHUNK,), jnp.float32),
                pltpu.VMEM((_CHUNK,), jnp.int32),
                pltpu.VMEM((_CHUNK,), jnp.float32),
                pltpu.SemaphoreType.DMA,
                pltpu.SemaphoreType.DMA,
            ]
        ),
    )
    return f(hP, rc, val)


def kernel(edge_index, edge_vals, u_f, v_f, W):
    node_f = jnp.concatenate([u_f, v_f], axis=0)
    hP = _matmul_pack(node_f, W.astype(jnp.float32))
    rc = _rc_pack(edge_index.astype(jnp.int32))
    outT = _spmm(hP, rc, edge_vals.astype(jnp.float32))
    return outT.T
